# baseline (speedup 1.0000x reference)
import dataclasses

import jax
import jax.numpy as jnp
from jax import lax
from jax.experimental import pallas as pl
from jax.experimental.pallas import tpu as pltpu
from jax.experimental.pallas import tpu_sc as plsc

VOCAB = 1000000
EMBED = 64
B = 4096
L = 50

NC = 2
NS = 16
NW = NC * NS
BW = B // NW
RW = BW * L
CB = 4
CHUNK = CB * L
NCHUNK = BW // CB
NPAIR = NCHUNK // 2
LPAD = 64
NG = L // 16
SSTRIDE = 17
TRASH = BW * LPAD
PAIRS = 128


def _sc_kernel(cpair_hbm, cpar_hbm, upair_hbm, upar_hbm, ev_hbm, eu_hbm,
               out_hbm, cpair_v, cpar_v, v_rows, upair_v, upar_v, u0, u1,
               s_tile, o_all, sem_v, sem0, sem1):
    wid = lax.axis_index("s") * NC + lax.axis_index("c")
    iota = lax.iota(jnp.int32, 16)

    pltpu.sync_copy(cpair_hbm.at[pl.ds(wid * BW, BW)], cpair_v)
    pltpu.sync_copy(cpar_hbm.at[pl.ds(wid * BW, BW)], cpar_v)
    pltpu.sync_copy(upair_hbm.at[pl.ds(wid * RW, RW)], upair_v)
    pltpu.sync_copy(upar_hbm.at[pl.ds(wid * RW, RW)], upar_v)

    pltpu.async_copy(ev_hbm.at[cpair_v], v_rows, sem_v).wait()

    def dot_row(u_rows, r, ob, v0, v1, v2, v3):
        rv = iota * 0 + r
        acc = plsc.load_gather(u_rows, [rv, ob]) * v0
        acc = acc + plsc.load_gather(u_rows, [rv, ob + 16]) * v1
        acc = acc + plsc.load_gather(u_rows, [rv, ob + 32]) * v2
        acc = acc + plsc.load_gather(u_rows, [rv, ob + 48]) * v3
        return acc

    def reduce_tile():
        out16 = s_tile[pl.ds(0, 16)]
        for k in range(1, 16):
            out16 = out16 + s_tile[pl.ds(SSTRIDE * k, 16)]
        return out16

    def compute_chunk(u_rows, c):
        def v_quarters(bb):
            bbv = iota * 0 + bb
            vo = plsc.load_gather(cpar_v, [bbv]) + iota
            v0 = plsc.load_gather(v_rows, [bbv, vo])
            v1 = plsc.load_gather(v_rows, [bbv, vo + 16])
            v2 = plsc.load_gather(v_rows, [bbv, vo + 32])
            v3 = plsc.load_gather(v_rows, [bbv, vo + 48])
            return v0, v1, v2, v3

        def u_off(r):
            return plsc.load_gather(upar_v, [iota * 0 + (c * CHUNK + r)]) + iota

        for b in range(CB):
            bb = c * CB + b
            v0, v1, v2, v3 = v_quarters(bb)
            for g in range(NG):
                for j in range(16):
                    r = b * L + 16 * g + j
                    acc = dot_row(u_rows, r, u_off(r), v0, v1, v2, v3)
                    plsc.store_scatter(s_tile, [iota * SSTRIDE + j], acc)
                o16 = reduce_tile()
                o_all[pl.ds(bb * LPAD + 16 * g, 16)] = o16

        for j in range(8):
            b = j // 2
            if j % 2 == 0:
                tv0, tv1, tv2, tv3 = v_quarters(c * CB + b)
            r = b * L + 48 + (j % 2)
            acc = dot_row(u_rows, r, u_off(r), tv0, tv1, tv2, tv3)
            plsc.store_scatter(s_tile, [iota * SSTRIDE + j], acc)
        o16 = reduce_tile()
        dest = jnp.where(
            iota < 8,
            (c * CB + iota // 2) * LPAD + 48 + (iota % 2),
            TRASH + iota,
        )
        plsc.store_scatter(o_all, [dest], o16)

    def gather_chunk(c, buf, sem):
        return pltpu.async_copy(
            eu_hbm.at[upair_v.at[pl.ds(c * CHUNK, CHUNK)]], buf, sem
        )

    gather_chunk(0, u0, sem0)

    @pl.loop(0, NPAIR)
    def _(g):
        a = 2 * g
        pltpu.make_async_copy(
            eu_hbm.at[upair_v.at[pl.ds(a * CHUNK, CHUNK)]], u0, sem0
        ).wait()
        gather_chunk(a + 1, u1, sem1)
        compute_chunk(u0, a)
        nxt = jnp.minimum(a + 2, NCHUNK - 1)
        gather_chunk(nxt, u0, sem0)
        pltpu.make_async_copy(
            eu_hbm.at[upair_v.at[pl.ds((a + 1) * CHUNK, CHUNK)]], u1, sem1
        ).wait()
        compute_chunk(u1, a + 1)

    pltpu.make_async_copy(
        eu_hbm.at[upair_v.at[pl.ds((NCHUNK - 1) * CHUNK, CHUNK)]], u0, sem0
    ).wait()

    pltpu.sync_copy(o_all.at[pl.ds(0, BW * LPAD)],
                    out_hbm.at[pl.ds(wid * BW * LPAD, BW * LPAD)])


def kernel(center, context_negative, embed_v, embed_u):
    crow = center.reshape(B)
    urow = context_negative.reshape(B * L)
    cpair = lax.shift_right_logical(crow, 1)
    cpar = (crow & 1) * EMBED
    upair = lax.shift_right_logical(urow, 1)
    upar = (urow & 1) * EMBED
    ev2 = embed_v.reshape(VOCAB // 2, PAIRS)
    eu2 = embed_u.reshape(VOCAB // 2, PAIRS)

    mesh = plsc.VectorSubcoreMesh(core_axis_name="c", subcore_axis_name="s")
    cp = pltpu.CompilerParams()
    fields = pltpu.CompilerParams.__dataclass_fields__
    if "needs_layout_passes" in fields:
        cp = dataclasses.replace(cp, needs_layout_passes=False)
    k = pl.kernel(
        _sc_kernel,
        compiler_params=cp,
        out_type=jax.ShapeDtypeStruct((B * LPAD,), jnp.float32),
        mesh=mesh,
        scratch_types=[
            pltpu.VMEM((BW,), jnp.int32),
            pltpu.VMEM((BW,), jnp.int32),
            pltpu.VMEM((BW, PAIRS), jnp.float32),
            pltpu.VMEM((RW,), jnp.int32),
            pltpu.VMEM((RW,), jnp.int32),
            pltpu.VMEM((CHUNK, PAIRS), jnp.float32),
            pltpu.VMEM((CHUNK, PAIRS), jnp.float32),
            pltpu.VMEM((SSTRIDE * 16,), jnp.float32),
            pltpu.VMEM((BW * LPAD + 16,), jnp.float32),
            pltpu.SemaphoreType.DMA,
            pltpu.SemaphoreType.DMA,
            pltpu.SemaphoreType.DMA,
        ],
    )
    out = k(cpair, cpar, upair, upar, ev2, eu2)
    return out.reshape(B, LPAD)[:, :L].reshape(B, 1, L)


# speedup vs baseline: 1.0954x; 1.0954x over previous
import dataclasses

import jax
import jax.numpy as jnp
from jax import lax
from jax.experimental import pallas as pl
from jax.experimental.pallas import tpu as pltpu
from jax.experimental.pallas import tpu_sc as plsc

VOCAB = 1000000
EMBED = 64
B = 4096
L = 50

NC = 2
NS = 16
NW = NC * NS
BW = B // NW
RW = BW * L
CB = 4
CHUNK = CB * L
NCHUNK = BW // CB
NPAIR = NCHUNK // 2
LPAD = 64
NG = L // 16
SSTRIDE = 17
TRASH = BW * LPAD
LAST = NCHUNK - 1


def _sc_kernel(crow_hbm, urow_hbm, ev_hbm, eu_hbm,
               out_hbm, crow_v, v_rows, i0, i1, u0, u1,
               s_tile, o_all, sem_v, sem0, sem1, isem0, isem1):
    wid = lax.axis_index("s") * NC + lax.axis_index("c")
    iota = lax.iota(jnp.int32, 16)

    pltpu.sync_copy(crow_hbm.at[pl.ds(wid * BW, BW)], crow_v)
    pltpu.async_copy(ev_hbm.at[crow_v], v_rows, sem_v).wait()

    def dot_row(u_rows, r, v0, v1, v2, v3):
        acc = u_rows[r, pl.ds(0, 16)] * v0
        acc = acc + u_rows[r, pl.ds(16, 16)] * v1
        acc = acc + u_rows[r, pl.ds(32, 16)] * v2
        acc = acc + u_rows[r, pl.ds(48, 16)] * v3
        return acc

    def reduce_tile():
        out16 = s_tile[pl.ds(0, 16)]
        for k in range(1, 16):
            out16 = out16 + s_tile[pl.ds(SSTRIDE * k, 16)]
        return out16

    def compute_chunk(u_rows, c):
        for b in range(CB):
            bb = c * CB + b
            v0 = v_rows[bb, pl.ds(0, 16)]
            v1 = v_rows[bb, pl.ds(16, 16)]
            v2 = v_rows[bb, pl.ds(32, 16)]
            v3 = v_rows[bb, pl.ds(48, 16)]
            for g in range(NG):
                for j in range(16):
                    r = b * L + 16 * g + j
                    acc = dot_row(u_rows, r, v0, v1, v2, v3)
                    plsc.store_scatter(s_tile, [iota * SSTRIDE + j], acc)
                o16 = reduce_tile()
                o_all[pl.ds(bb * LPAD + 16 * g, 16)] = o16

        for j in range(8):
            b = j // 2
            if j % 2 == 0:
                tv0 = v_rows[c * CB + b, pl.ds(0, 16)]
                tv1 = v_rows[c * CB + b, pl.ds(16, 16)]
                tv2 = v_rows[c * CB + b, pl.ds(32, 16)]
                tv3 = v_rows[c * CB + b, pl.ds(48, 16)]
            r = b * L + 48 + (j % 2)
            acc = dot_row(u_rows, r, tv0, tv1, tv2, tv3)
            plsc.store_scatter(s_tile, [iota * SSTRIDE + j], acc)
        o16 = reduce_tile()
        dest = jnp.where(
            iota < 8,
            (c * CB + iota // 2) * LPAD + 48 + (iota % 2),
            TRASH + iota,
        )
        plsc.store_scatter(o_all, [dest], o16)

    def stage_idx(c, ibuf, isem):
        return pltpu.async_copy(
            urow_hbm.at[pl.ds(wid * RW + c * CHUNK, CHUNK)], ibuf, isem
        )

    def wait_idx(c, ibuf, isem):
        pltpu.make_async_copy(
            urow_hbm.at[pl.ds(wid * RW + c * CHUNK, CHUNK)], ibuf, isem
        ).wait()

    def gather_chunk(ibuf, buf, sem):
        return pltpu.async_copy(eu_hbm.at[ibuf], buf, sem)

    def wait_gather(ibuf, buf, sem):
        pltpu.make_async_copy(eu_hbm.at[ibuf], buf, sem).wait()

    stage_idx(0, i0, isem0).wait()
    gather_chunk(i0, u0, sem0)
    stage_idx(1, i1, isem1)

    @pl.loop(0, NPAIR)
    def _(g):
        a = 2 * g
        wait_gather(i0, u0, sem0)
        wait_idx(a + 1, i1, isem1)
        gather_chunk(i1, u1, sem1)
        stage_idx(jnp.minimum(a + 2, LAST), i0, isem0)
        compute_chunk(u0, a)
        wait_idx(jnp.minimum(a + 2, LAST), i0, isem0)
        gather_chunk(i0, u0, sem0)
        wait_gather(i1, u1, sem1)
        stage_idx(jnp.minimum(a + 3, LAST), i1, isem1)
        compute_chunk(u1, a + 1)

    wait_gather(i0, u0, sem0)
    wait_idx(LAST, i1, isem1)

    pltpu.sync_copy(o_all.at[pl.ds(0, BW * LPAD)],
                    out_hbm.at[pl.ds(wid * BW * LPAD, BW * LPAD)])


def kernel(center, context_negative, embed_v, embed_u):
    crow = center.reshape(B)
    urow = context_negative.reshape(B * L)

    mesh = plsc.VectorSubcoreMesh(core_axis_name="c", subcore_axis_name="s")
    cp = pltpu.CompilerParams()
    fields = pltpu.CompilerParams.__dataclass_fields__
    if "needs_layout_passes" in fields:
        cp = dataclasses.replace(cp, needs_layout_passes=False)
    if "use_tc_tiling_on_sc" in fields:
        cp = dataclasses.replace(cp, use_tc_tiling_on_sc=False)
    k = pl.kernel(
        _sc_kernel,
        compiler_params=cp,
        out_type=jax.ShapeDtypeStruct((B * LPAD,), jnp.float32),
        mesh=mesh,
        scratch_types=[
            pltpu.VMEM((BW,), jnp.int32),
            pltpu.VMEM((BW, EMBED), jnp.float32),
            pltpu.VMEM((CHUNK,), jnp.int32),
            pltpu.VMEM((CHUNK,), jnp.int32),
            pltpu.VMEM((CHUNK, EMBED), jnp.float32),
            pltpu.VMEM((CHUNK, EMBED), jnp.float32),
            pltpu.VMEM((SSTRIDE * 16,), jnp.float32),
            pltpu.VMEM((BW * LPAD + 16,), jnp.float32),
            pltpu.SemaphoreType.DMA,
            pltpu.SemaphoreType.DMA,
            pltpu.SemaphoreType.DMA,
            pltpu.SemaphoreType.DMA,
            pltpu.SemaphoreType.DMA,
        ],
    )
    out = k(crow, urow, embed_v, embed_u)
    return out.reshape(B, LPAD)[:, :L].reshape(B, 1, L)


# speedup vs baseline: 1.0970x; 1.0015x over previous
import dataclasses

import jax
import jax.numpy as jnp
from jax import lax
from jax.experimental import pallas as pl
from jax.experimental.pallas import tpu as pltpu
from jax.experimental.pallas import tpu_sc as plsc

VOCAB = 1000000
EMBED = 64
B = 4096
L = 50

NC = 2
NS = 16
NW = NC * NS
BW = B // NW
RW = BW * L
CB = 4
CHUNK = CB * L
NCHUNK = BW // CB
NPAIR = NCHUNK // 2
LPAD = 64
NG = L // 16
SSTRIDE = 17
TRASH = BW * LPAD
LAST = NCHUNK - 1


def _sc_kernel(crow_hbm, urow_hbm, eu_hbm, ev_hbm,
               out_hbm, crow_v, v_rows, i0, i1, u0, u1,
               s_tile, o_all, sem_v, sem0, sem1, isem0, isem1):
    wid = lax.axis_index("s") * NC + lax.axis_index("c")
    iota = lax.iota(jnp.int32, 16)

    pltpu.sync_copy(crow_hbm.at[pl.ds(wid * BW, BW)], crow_v)
    pltpu.async_copy(ev_hbm.at[crow_v], v_rows, sem_v).wait()

    def dot_row(u_rows, r, v0, v1, v2, v3):
        acc = u_rows[r, pl.ds(0, 16)] * v0
        acc = acc + u_rows[r, pl.ds(16, 16)] * v1
        acc = acc + u_rows[r, pl.ds(32, 16)] * v2
        acc = acc + u_rows[r, pl.ds(48, 16)] * v3
        return acc

    def reduce_tile():
        out16 = s_tile[pl.ds(0, 16)]
        for k in range(1, 16):
            out16 = out16 + s_tile[pl.ds(SSTRIDE * k, 16)]
        return out16

    def compute_chunk(u_rows, c):
        for b in range(CB):
            bb = c * CB + b
            v0 = v_rows[bb, pl.ds(0, 16)]
            v1 = v_rows[bb, pl.ds(16, 16)]
            v2 = v_rows[bb, pl.ds(32, 16)]
            v3 = v_rows[bb, pl.ds(48, 16)]
            for g in range(NG):
                for j in range(16):
                    r = b * L + 16 * g + j
                    acc = dot_row(u_rows, r, v0, v1, v2, v3)
                    plsc.store_scatter(s_tile, [iota * SSTRIDE + j], acc)
                o16 = reduce_tile()
                o_all[pl.ds(bb * LPAD + 16 * g, 16)] = o16

        for j in range(8):
            b = j // 2
            if j % 2 == 0:
                tv0 = v_rows[c * CB + b, pl.ds(0, 16)]
                tv1 = v_rows[c * CB + b, pl.ds(16, 16)]
                tv2 = v_rows[c * CB + b, pl.ds(32, 16)]
                tv3 = v_rows[c * CB + b, pl.ds(48, 16)]
            r = b * L + 48 + (j % 2)
            acc = dot_row(u_rows, r, tv0, tv1, tv2, tv3)
            plsc.store_scatter(s_tile, [iota * SSTRIDE + j], acc)
        o16 = reduce_tile()
        dest = jnp.where(
            iota < 8,
            (c * CB + iota // 2) * LPAD + 48 + (iota % 2),
            TRASH + iota,
        )
        plsc.store_scatter(o_all, [dest], o16)

    def stage_idx(c, ibuf, isem):
        return pltpu.async_copy(
            urow_hbm.at[pl.ds(wid * RW + c * CHUNK, CHUNK)], ibuf, isem
        )

    def wait_idx(c, ibuf, isem):
        pltpu.make_async_copy(
            urow_hbm.at[pl.ds(wid * RW + c * CHUNK, CHUNK)], ibuf, isem
        ).wait()

    def gather_chunk(ibuf, buf, sem):
        return pltpu.async_copy(eu_hbm.at[ibuf], buf, sem)

    def wait_gather(ibuf, buf, sem):
        pltpu.make_async_copy(eu_hbm.at[ibuf], buf, sem).wait()

    stage_idx(0, i0, isem0).wait()
    gather_chunk(i0, u0, sem0)
    stage_idx(1, i1, isem1)

    @pl.loop(0, NPAIR)
    def _(g):
        a = 2 * g
        wait_gather(i0, u0, sem0)
        wait_idx(a + 1, i1, isem1)
        gather_chunk(i1, u1, sem1)
        stage_idx(jnp.minimum(a + 2, LAST), i0, isem0)
        compute_chunk(u0, a)
        wait_idx(jnp.minimum(a + 2, LAST), i0, isem0)
        gather_chunk(i0, u0, sem0)
        wait_gather(i1, u1, sem1)
        stage_idx(jnp.minimum(a + 3, LAST), i1, isem1)
        compute_chunk(u1, a + 1)

    wait_gather(i0, u0, sem0)
    wait_idx(LAST, i1, isem1)

    pltpu.sync_copy(o_all.at[pl.ds(0, BW * LPAD)],
                    out_hbm.at[pl.ds(wid * BW * LPAD, BW * LPAD)])


def kernel(center, context_negative, embed_v, embed_u):
    crow = center.reshape(B)
    urow = context_negative.reshape(B * L)

    mesh = plsc.VectorSubcoreMesh(core_axis_name="c", subcore_axis_name="s")
    cp = pltpu.CompilerParams()
    fields = pltpu.CompilerParams.__dataclass_fields__
    if "needs_layout_passes" in fields:
        cp = dataclasses.replace(cp, needs_layout_passes=False)
    if "use_tc_tiling_on_sc" in fields:
        cp = dataclasses.replace(cp, use_tc_tiling_on_sc=False)
    k = pl.kernel(
        _sc_kernel,
        compiler_params=cp,
        out_type=jax.ShapeDtypeStruct((B * LPAD,), jnp.float32),
        mesh=mesh,
        scratch_types=[
            pltpu.VMEM((BW,), jnp.int32),
            pltpu.VMEM((BW, EMBED), jnp.float32),
            pltpu.VMEM((CHUNK,), jnp.int32),
            pltpu.VMEM((CHUNK,), jnp.int32),
            pltpu.VMEM((CHUNK, EMBED), jnp.float32),
            pltpu.VMEM((CHUNK, EMBED), jnp.float32),
            pltpu.VMEM((SSTRIDE * 16,), jnp.float32),
            pltpu.VMEM((BW * LPAD + 16,), jnp.float32),
            pltpu.SemaphoreType.DMA,
            pltpu.SemaphoreType.DMA,
            pltpu.SemaphoreType.DMA,
            pltpu.SemaphoreType.DMA,
            pltpu.SemaphoreType.DMA,
        ],
    )
    out = k(crow, urow, embed_u, embed_v)
    return out.reshape(B, LPAD)[:, :L].reshape(B, 1, L)
